# baseline (speedup 1.0000x reference)
import jax
import jax.numpy as jnp
from jax.experimental import pallas as pl
from jax.experimental.pallas import tpu as pltpu

_D = 1024
_E = 16
_H = 128
_K = 2
_BM = 128
_N = 4096
_R = _N * _K
_P = _R + _E * _BM
_NT = _P // _BM


def _router_kernel(x_ref, selt_ref, gates_ref):
    logits = jnp.dot(x_ref[...], selt_ref[...], preferred_element_type=jnp.float32)
    gates_ref[...] = jax.nn.sigmoid(logits)


def _ffn_kernel(eot_ref, xs_ref, gate_ref, keys_ref, values_ref, out_ref):
    del eot_ref
    h = jnp.dot(xs_ref[...], keys_ref[0], preferred_element_type=jnp.float32)
    h = jnp.maximum(h, 0.0) * gate_ref[0, 0][:, None]
    out_ref[...] = jnp.dot(h, values_ref[0], preferred_element_type=jnp.float32)


def kernel(x, expert_sel, keys, values):
    gates = pl.pallas_call(
        _router_kernel,
        out_shape=jax.ShapeDtypeStruct((_N, _E), jnp.float32),
    )(x, expert_sel.T)

    topv, topi = jax.lax.top_k(gates, _K)
    eids = topi.reshape(-1).astype(jnp.int32)
    gvals = topv.reshape(-1)
    tok = jnp.repeat(jnp.arange(_N, dtype=jnp.int32), _K)

    oh = (eids[:, None] == jnp.arange(_E, dtype=jnp.int32)[None, :]).astype(jnp.int32)
    ranks = jnp.take_along_axis(jnp.cumsum(oh, axis=0) - oh, eids[:, None], axis=1)[:, 0]
    counts = jnp.sum(oh, axis=0)
    padded_counts = ((counts + _BM - 1) // _BM) * _BM
    ends = jnp.cumsum(padded_counts)
    pad_off = ends - padded_counts
    slot = pad_off[eids] + ranks

    tok_pad = jnp.zeros((_P,), jnp.int32).at[slot].set(tok)
    gate_pad = jnp.zeros((_P,), jnp.float32).at[slot].set(gvals)
    xs_pad = x[tok_pad]

    tile_starts = jnp.arange(_NT, dtype=jnp.int32) * _BM
    eot = jnp.minimum(
        jnp.searchsorted(ends, tile_starts, side="right").astype(jnp.int32), _E - 1)

    ys = pl.pallas_call(
        _ffn_kernel,
        grid_spec=pltpu.PrefetchScalarGridSpec(
            num_scalar_prefetch=1,
            grid=(_NT,),
            in_specs=[
                pl.BlockSpec((_BM, _D), lambda i, eot: (i, 0)),
                pl.BlockSpec((1, 1, _BM), lambda i, eot: (i, 0, 0)),
                pl.BlockSpec((1, _D, _H), lambda i, eot: (eot[i], 0, 0)),
                pl.BlockSpec((1, _H, _D), lambda i, eot: (eot[i], 0, 0)),
            ],
            out_specs=pl.BlockSpec((_BM, _D), lambda i, eot: (i, 0)),
        ),
        out_shape=jax.ShapeDtypeStruct((_P, _D), jnp.float32),
    )(eot, xs_pad, gate_pad.reshape(_NT, 1, _BM), keys, values)

    return jnp.sum(ys[slot].reshape(_N, _K, _D), axis=1)
